# Initial kernel scaffold; baseline (speedup 1.0000x reference)
import jax
import jax.numpy as jnp
from jax.experimental import pallas as pl


def kernel(predicted_labels, predicted_offsets, gt_boxes, anchors):
    raise NotImplementedError("write your pallas kernel here")


# speedup vs baseline: 38.9243x; 38.9243x over previous
import functools

import jax
import jax.numpy as jnp
from jax.experimental import pallas as pl
from jax.experimental.pallas import tpu as pltpu

_ANCHOR_THRESHOLD = 0.3
_LANES = 128


def _box_loss_kernel(gt_ref, labels_ref, off_ref, anc_ref, out_ref, *, B, A, M):
    R = A // _LANES
    f32 = jnp.float32

    ax0 = anc_ref[0]
    ay0 = anc_ref[1]
    ax1 = anc_ref[2]
    ay1 = anc_ref[3]
    area_a = (ax1 - ax0) * (ay1 - ay0)
    pcx = (ax0 + ax1) / 2.0
    pcy = (ay0 + ay1) / 2.0
    pw = ax1 - ax0
    ph = ay1 - ay0
    pw10 = pw / 10.0
    ph10 = ph / 10.0

    row_i = jax.lax.broadcasted_iota(jnp.int32, (R, _LANES), 0)
    col_i = jax.lax.broadcasted_iota(jnp.int32, (R, _LANES), 1)
    flat_iota = (row_i * _LANES + col_i).astype(f32)

    def body(b, carry):
        n_pos_tot, box_num, cls_pos_sum, cls_hard_sum = carry

        best = jnp.full((R, _LANES), -1.0, f32)
        gx0s = jnp.zeros((R, _LANES), f32)
        gy0s = jnp.zeros((R, _LANES), f32)
        gx1s = jnp.zeros((R, _LANES), f32)
        gy1s = jnp.zeros((R, _LANES), f32)
        gts = []
        afos = []
        for m in range(M):
            gx0 = gt_ref[b, m, 0]
            gy0 = gt_ref[b, m, 1]
            gx1 = gt_ref[b, m, 2]
            gy1 = gt_ref[b, m, 3]
            gts.append((gx0, gy0, gx1, gy1))
            iw = jnp.maximum(jnp.minimum(ax1, gx1) - jnp.maximum(ax0, gx0), 0.0)
            ih = jnp.maximum(jnp.minimum(ay1, gy1) - jnp.maximum(ay0, gy0), 0.0)
            inter = iw * ih
            area_b = (gx1 - gx0) * (gy1 - gy0)
            iou = inter / (area_a + area_b - inter)
            upd = iou > best
            best = jnp.where(upd, iou, best)
            gx0s = jnp.where(upd, gx0, gx0s)
            gy0s = jnp.where(upd, gy0, gy0s)
            gx1s = jnp.where(upd, gx1, gx1s)
            gy1s = jnp.where(upd, gy1, gy1s)
            cmax = jnp.max(iou)
            afos.append(jnp.min(jnp.where(iou == cmax, flat_iota, 1e9)))

        for m in range(M):
            mask = flat_iota == afos[m]
            gx0, gy0, gx1, gy1 = gts[m]
            best = jnp.where(mask, 1.0, best)
            gx0s = jnp.where(mask, gx0, gx0s)
            gy0s = jnp.where(mask, gy0, gy0s)
            gx1s = jnp.where(mask, gx1, gx1s)
            gy1s = jnp.where(mask, gy1, gy1s)

        pos = best > _ANCHOR_THRESHOLD
        n_pos = jnp.sum(pos.astype(f32))

        cx = (gx0s + gx1s) / 2.0
        cy = (gy0s + gy1s) / 2.0
        w = gx1s - gx0s
        h = gy1s - gy0s
        tx = (cx - pcx) / pw10
        ty = (cy - pcy) / ph10
        tw = jnp.log(jnp.maximum(w, 1e-8) / pw) * 5.0
        th = jnp.log(jnp.maximum(h, 1e-8) / ph) * 5.0
        bnum = box_num
        for c, t in enumerate((tx, ty, tw, th)):
            d = off_ref[b, c] - t
            ad = jnp.abs(d)
            e = jnp.where(ad < 1.0, 0.5 * d * d, ad - 0.5)
            bnum = bnum + jnp.sum(jnp.where(pos, e, 0.0))

        x = labels_ref[b]
        sp = jnp.maximum(x, 0.0) + jnp.log1p(jnp.exp(-jnp.abs(x)))
        cps = cls_pos_sum + jnp.sum(jnp.where(pos, sp - x, 0.0))
        v = jnp.where(pos, 0.0, sp)

        k = 3.0 * n_pos
        vb = jax.lax.bitcast_convert_type(v, jnp.int32)
        T = jnp.int32(0)
        for bit in range(30, -1, -1):
            cand = T | jnp.int32(1 << bit)
            cnt = jnp.sum((vb >= cand).astype(f32))
            T = jnp.where(cnt >= k, cand, T)
        Tval = jax.lax.bitcast_convert_type(T, f32)
        gt_mask = vb > T
        cntgt = jnp.sum(gt_mask.astype(f32))
        sumgt = jnp.sum(jnp.where(gt_mask, v, 0.0))
        hard_b = sumgt + (k - cntgt) * Tval
        chs = cls_hard_sum + jnp.where(k > 0.0, hard_b, 0.0)

        return n_pos_tot + n_pos, bnum, cps, chs

    init = (jnp.float32(0.0),) * 4
    n_pos_tot, box_num, cls_pos_sum, cls_hard_sum = jax.lax.fori_loop(
        0, B, body, init)

    box_loss = box_num / (4.0 * n_pos_tot)
    cls_loss = (cls_hard_sum + cls_pos_sum) / n_pos_tot
    out_ref[0] = box_loss + cls_loss
    out_ref[1] = box_loss
    out_ref[2] = cls_loss


@jax.jit
def kernel(predicted_labels, predicted_offsets, gt_boxes, anchors):
    B, A, _ = predicted_labels.shape
    M = gt_boxes.shape[1]
    R = A // _LANES
    labels = predicted_labels.reshape(B, R, _LANES)
    off_t = predicted_offsets.transpose(0, 2, 1).reshape(B, 4, R, _LANES)
    anc_t = anchors.T.reshape(4, R, _LANES)

    out = pl.pallas_call(
        functools.partial(_box_loss_kernel, B=B, A=A, M=M),
        out_shape=jax.ShapeDtypeStruct((3,), jnp.float32),
        in_specs=[
            pl.BlockSpec(memory_space=pltpu.SMEM),
            pl.BlockSpec(memory_space=pltpu.VMEM),
            pl.BlockSpec(memory_space=pltpu.VMEM),
            pl.BlockSpec(memory_space=pltpu.VMEM),
        ],
        out_specs=pl.BlockSpec(memory_space=pltpu.SMEM),
    )(gt_boxes, labels, off_t, anc_t)
    return (out[0], out[1], out[2])


# speedup vs baseline: 71.1770x; 1.8286x over previous
import functools

import jax
import jax.numpy as jnp
from jax.experimental import pallas as pl
from jax.experimental.pallas import tpu as pltpu

_ANCHOR_THRESHOLD = 0.3
_CHUNK = 8


def _box_loss_kernel(gt_ref, labels_ref, off_ref, anc_ref, out_ref, acc_ref,
                     *, A, M, n_chunks):
    f32 = jnp.float32
    C = _CHUNK
    step = pl.program_id(0)

    @pl.when(step == 0)
    def _():
        acc_ref[0] = 0.0
        acc_ref[1] = 0.0
        acc_ref[2] = 0.0
        acc_ref[3] = 0.0

    ax0 = anc_ref[0]
    ay0 = anc_ref[1]
    ax1 = anc_ref[2]
    ay1 = anc_ref[3]
    area_a = (ax1 - ax0) * (ay1 - ay0)
    pcx = (ax0 + ax1) / 2.0
    pcy = (ay0 + ay1) / 2.0
    pw = ax1 - ax0
    ph = ay1 - ay0
    pw10 = pw / 10.0
    ph10 = ph / 10.0

    flat_iota = jax.lax.broadcasted_iota(jnp.int32, (C, A), 1).astype(f32)

    best = jnp.full((C, A), -1.0, f32)
    gx0s = jnp.zeros((C, A), f32)
    gy0s = jnp.zeros((C, A), f32)
    gx1s = jnp.zeros((C, A), f32)
    gy1s = jnp.zeros((C, A), f32)
    gts = []
    afos = []
    for m in range(M):
        gx0 = gt_ref[4 * m + 0]
        gy0 = gt_ref[4 * m + 1]
        gx1 = gt_ref[4 * m + 2]
        gy1 = gt_ref[4 * m + 3]
        gts.append((gx0, gy0, gx1, gy1))
        iw = jnp.maximum(jnp.minimum(ax1, gx1) - jnp.maximum(ax0, gx0), 0.0)
        ih = jnp.maximum(jnp.minimum(ay1, gy1) - jnp.maximum(ay0, gy0), 0.0)
        inter = iw * ih
        area_b = (gx1 - gx0) * (gy1 - gy0)
        iou = inter / (area_a + area_b - inter)
        upd = iou > best
        best = jnp.where(upd, iou, best)
        gx0s = jnp.where(upd, gx0, gx0s)
        gy0s = jnp.where(upd, gy0, gy0s)
        gx1s = jnp.where(upd, gx1, gx1s)
        gy1s = jnp.where(upd, gy1, gy1s)
        cmax = jnp.max(iou, axis=1, keepdims=True)
        afos.append(jnp.min(jnp.where(iou == cmax, flat_iota, 1e9),
                            axis=1, keepdims=True))

    for m in range(M):
        mask = flat_iota == afos[m]
        gx0, gy0, gx1, gy1 = gts[m]
        best = jnp.where(mask, 1.0, best)
        gx0s = jnp.where(mask, gx0, gx0s)
        gy0s = jnp.where(mask, gy0, gy0s)
        gx1s = jnp.where(mask, gx1, gx1s)
        gy1s = jnp.where(mask, gy1, gy1s)

    pos = best > _ANCHOR_THRESHOLD
    n_pos_v = jnp.sum(pos.astype(f32), axis=1, keepdims=True)

    cx = (gx0s + gx1s) / 2.0
    cy = (gy0s + gy1s) / 2.0
    w = gx1s - gx0s
    h = gy1s - gy0s
    tx = (cx - pcx) / pw10
    ty = (cy - pcy) / ph10
    tw = jnp.log(jnp.maximum(w, 1e-8) / pw) * 5.0
    th = jnp.log(jnp.maximum(h, 1e-8) / ph) * 5.0
    esum = jnp.zeros((C, A), f32)
    for c, t in enumerate((tx, ty, tw, th)):
        d = off_ref[c] - t
        ad = jnp.abs(d)
        esum = esum + jnp.where(ad < 1.0, 0.5 * d * d, ad - 0.5)
    box_num = jnp.sum(jnp.where(pos, esum, 0.0))

    x = labels_ref[...]
    sp = jnp.maximum(x, 0.0) + jnp.log1p(jnp.exp(-jnp.abs(x)))
    cls_pos = jnp.sum(jnp.where(pos, sp - x, 0.0))
    v = jnp.where(pos, 0.0, sp)

    k = 3.0 * n_pos_v
    vb = jax.lax.bitcast_convert_type(v, jnp.int32)
    T = jnp.zeros((C, 1), jnp.int32)
    for bit in range(30, -1, -1):
        cand = T | jnp.int32(1 << bit)
        cnt = jnp.sum((vb >= cand).astype(f32), axis=1, keepdims=True)
        T = jnp.where(cnt >= k, cand, T)
    Tval = jax.lax.bitcast_convert_type(T, f32)
    gt_mask = vb > T
    cntgt = jnp.sum(gt_mask.astype(f32), axis=1, keepdims=True)
    sumgt = jnp.sum(jnp.where(gt_mask, v, 0.0), axis=1, keepdims=True)
    hard_v = sumgt + (k - cntgt) * Tval
    cls_hard = jnp.sum(jnp.where(k > 0.0, hard_v, 0.0))

    acc_ref[0] = acc_ref[0] + jnp.sum(n_pos_v)
    acc_ref[1] = acc_ref[1] + box_num
    acc_ref[2] = acc_ref[2] + cls_pos
    acc_ref[3] = acc_ref[3] + cls_hard

    @pl.when(step == n_chunks - 1)
    def _():
        n_pos_tot = acc_ref[0]
        box_loss = acc_ref[1] / (4.0 * n_pos_tot)
        cls_loss = (acc_ref[3] + acc_ref[2]) / n_pos_tot
        out_ref[0] = box_loss + cls_loss
        out_ref[1] = box_loss
        out_ref[2] = cls_loss


@jax.jit
def kernel(predicted_labels, predicted_offsets, gt_boxes, anchors):
    B, A, _ = predicted_labels.shape
    M = gt_boxes.shape[1]
    C = _CHUNK
    n_chunks = B // C
    labels = predicted_labels.reshape(B, A)
    off_t = predicted_offsets.transpose(2, 0, 1)
    gt_t = gt_boxes.transpose(1, 2, 0).reshape(4 * M, B, 1)
    anc_t = anchors.T

    out = pl.pallas_call(
        functools.partial(_box_loss_kernel, A=A, M=M, n_chunks=n_chunks),
        grid=(n_chunks,),
        out_shape=jax.ShapeDtypeStruct((3,), jnp.float32),
        in_specs=[
            pl.BlockSpec((4 * M, C, 1), lambda i: (0, i, 0)),
            pl.BlockSpec((C, A), lambda i: (i, 0)),
            pl.BlockSpec((4, C, A), lambda i: (0, i, 0)),
            pl.BlockSpec((4, A), lambda i: (0, 0)),
        ],
        out_specs=pl.BlockSpec(memory_space=pltpu.SMEM),
        scratch_shapes=[pltpu.SMEM((4,), jnp.float32)],
    )(gt_t, labels, off_t, anc_t)
    return (out[0], out[1], out[2])


# speedup vs baseline: 72.7166x; 1.0216x over previous
import functools

import jax
import jax.numpy as jnp
from jax.experimental import pallas as pl
from jax.experimental.pallas import tpu as pltpu

_ANCHOR_THRESHOLD = 0.3
_CHUNK = 8


def _match_cls_kernel(gt_ref, labels_ref, anc_ref, t4_ref, pos_ref, acc_ref,
                      scr_ref, *, A, M, n_chunks):
    f32 = jnp.float32
    C = _CHUNK
    step = pl.program_id(0)

    @pl.when(step == 0)
    def _():
        scr_ref[0] = 0.0
        scr_ref[1] = 0.0
        scr_ref[2] = 0.0

    ax0 = anc_ref[0]
    ay0 = anc_ref[1]
    ax1 = anc_ref[2]
    ay1 = anc_ref[3]
    area_a = (ax1 - ax0) * (ay1 - ay0)
    pcx = (ax0 + ax1) / 2.0
    pcy = (ay0 + ay1) / 2.0
    pw = ax1 - ax0
    ph = ay1 - ay0
    pw10 = pw / 10.0
    ph10 = ph / 10.0

    flat_iota = jax.lax.broadcasted_iota(jnp.int32, (C, A), 1).astype(f32)

    best = jnp.full((C, A), -1.0, f32)
    gx0s = jnp.zeros((C, A), f32)
    gy0s = jnp.zeros((C, A), f32)
    gx1s = jnp.zeros((C, A), f32)
    gy1s = jnp.zeros((C, A), f32)
    gts = []
    afos = []
    for m in range(M):
        gx0 = gt_ref[4 * m + 0]
        gy0 = gt_ref[4 * m + 1]
        gx1 = gt_ref[4 * m + 2]
        gy1 = gt_ref[4 * m + 3]
        gts.append((gx0, gy0, gx1, gy1))
        iw = jnp.maximum(jnp.minimum(ax1, gx1) - jnp.maximum(ax0, gx0), 0.0)
        ih = jnp.maximum(jnp.minimum(ay1, gy1) - jnp.maximum(ay0, gy0), 0.0)
        inter = iw * ih
        area_b = (gx1 - gx0) * (gy1 - gy0)
        iou = inter / (area_a + area_b - inter)
        upd = iou > best
        best = jnp.where(upd, iou, best)
        gx0s = jnp.where(upd, gx0, gx0s)
        gy0s = jnp.where(upd, gy0, gy0s)
        gx1s = jnp.where(upd, gx1, gx1s)
        gy1s = jnp.where(upd, gy1, gy1s)
        cmax = jnp.max(iou, axis=1, keepdims=True)
        afos.append(jnp.min(jnp.where(iou == cmax, flat_iota, 1e9),
                            axis=1, keepdims=True))

    for m in range(M):
        mask = flat_iota == afos[m]
        gx0, gy0, gx1, gy1 = gts[m]
        best = jnp.where(mask, 1.0, best)
        gx0s = jnp.where(mask, gx0, gx0s)
        gy0s = jnp.where(mask, gy0, gy0s)
        gx1s = jnp.where(mask, gx1, gx1s)
        gy1s = jnp.where(mask, gy1, gy1s)

    pos = best > _ANCHOR_THRESHOLD
    pos_f = pos.astype(f32)
    pos_ref[...] = pos_f
    n_pos_v = jnp.sum(pos_f, axis=1, keepdims=True)

    cx = (gx0s + gx1s) / 2.0
    cy = (gy0s + gy1s) / 2.0
    w = gx1s - gx0s
    h = gy1s - gy0s
    t4_ref[0] = (cx - pcx) / pw10
    t4_ref[1] = (cy - pcy) / ph10
    t4_ref[2] = jnp.log(jnp.maximum(w, 1e-8) / pw) * 5.0
    t4_ref[3] = jnp.log(jnp.maximum(h, 1e-8) / ph) * 5.0

    x = labels_ref[...]
    sp = jnp.maximum(x, 0.0) + jnp.log1p(jnp.exp(-jnp.abs(x)))
    cls_pos = jnp.sum(jnp.where(pos, sp - x, 0.0))
    v = jnp.where(pos, 0.0, sp)

    k = 3.0 * n_pos_v
    vb = jax.lax.bitcast_convert_type(v, jnp.int32)
    T = jnp.zeros((C, 1), jnp.int32)
    for bit in range(30, -1, -1):
        cand = T | jnp.int32(1 << bit)
        cnt = jnp.sum((vb >= cand).astype(f32), axis=1, keepdims=True)
        T = jnp.where(cnt >= k, cand, T)
    Tval = jax.lax.bitcast_convert_type(T, f32)
    gt_mask = vb > T
    cntgt = jnp.sum(gt_mask.astype(f32), axis=1, keepdims=True)
    sumgt = jnp.sum(jnp.where(gt_mask, v, 0.0), axis=1, keepdims=True)
    hard_v = sumgt + (k - cntgt) * Tval
    cls_hard = jnp.sum(jnp.where(k > 0.0, hard_v, 0.0))

    scr_ref[0] = scr_ref[0] + jnp.sum(n_pos_v)
    scr_ref[1] = scr_ref[1] + cls_pos
    scr_ref[2] = scr_ref[2] + cls_hard

    @pl.when(step == n_chunks - 1)
    def _():
        acc_ref[0] = scr_ref[0]
        acc_ref[1] = scr_ref[1]
        acc_ref[2] = scr_ref[2]


def _box_kernel(off_ref, t4_ref, pos_ref, acc_ref, out_ref, scr_ref,
                *, n_chunks):
    step = pl.program_id(0)

    @pl.when(step == 0)
    def _():
        scr_ref[0] = 0.0

    pos = pos_ref[...] > 0.5
    esum = None
    for c in range(4):
        d = off_ref[c] - t4_ref[c]
        ad = jnp.abs(d)
        e = jnp.where(ad < 1.0, 0.5 * d * d, ad - 0.5)
        esum = e if esum is None else esum + e
    scr_ref[0] = scr_ref[0] + jnp.sum(jnp.where(pos, esum, 0.0))

    @pl.when(step == n_chunks - 1)
    def _():
        n_pos_tot = acc_ref[0]
        box_loss = scr_ref[0] / (4.0 * n_pos_tot)
        cls_loss = (acc_ref[2] + acc_ref[1]) / n_pos_tot
        out_ref[0] = box_loss + cls_loss
        out_ref[1] = box_loss
        out_ref[2] = cls_loss


@jax.jit
def kernel(predicted_labels, predicted_offsets, gt_boxes, anchors):
    B, A, _ = predicted_labels.shape
    M = gt_boxes.shape[1]
    C = _CHUNK
    n_chunks = B // C
    labels = predicted_labels.reshape(B, A)
    off_t = predicted_offsets.transpose(2, 0, 1)
    gt_t = gt_boxes.transpose(1, 2, 0).reshape(4 * M, B, 1)
    anc_t = anchors.T

    t4, posm, acc = pl.pallas_call(
        functools.partial(_match_cls_kernel, A=A, M=M, n_chunks=n_chunks),
        grid=(n_chunks,),
        out_shape=(
            jax.ShapeDtypeStruct((4, B, A), jnp.float32),
            jax.ShapeDtypeStruct((B, A), jnp.float32),
            jax.ShapeDtypeStruct((3,), jnp.float32),
        ),
        in_specs=[
            pl.BlockSpec((4 * M, C, 1), lambda i: (0, i, 0)),
            pl.BlockSpec((C, A), lambda i: (i, 0)),
            pl.BlockSpec((4, A), lambda i: (0, 0)),
        ],
        out_specs=(
            pl.BlockSpec((4, C, A), lambda i: (0, i, 0)),
            pl.BlockSpec((C, A), lambda i: (i, 0)),
            pl.BlockSpec(memory_space=pltpu.SMEM),
        ),
        scratch_shapes=[pltpu.SMEM((3,), jnp.float32)],
    )(gt_t, labels, anc_t)

    out = pl.pallas_call(
        functools.partial(_box_kernel, n_chunks=n_chunks),
        grid=(n_chunks,),
        out_shape=jax.ShapeDtypeStruct((3,), jnp.float32),
        in_specs=[
            pl.BlockSpec((4, C, A), lambda i: (0, i, 0)),
            pl.BlockSpec((4, C, A), lambda i: (0, i, 0)),
            pl.BlockSpec((C, A), lambda i: (i, 0)),
            pl.BlockSpec(memory_space=pltpu.SMEM),
        ],
        out_specs=pl.BlockSpec(memory_space=pltpu.SMEM),
        scratch_shapes=[pltpu.SMEM((1,), jnp.float32)],
    )(off_t, t4, posm, acc)
    return (out[0], out[1], out[2])
